# Initial kernel scaffold; baseline (speedup 1.0000x reference)
import jax
import jax.numpy as jnp
from jax.experimental import pallas as pl


def kernel(x, Wr, br, W1, b1, W2, b2, b_buf):
    raise NotImplementedError("write your pallas kernel here")


# speedup vs baseline: 2.3630x; 2.3630x over previous
import functools

import jax
import jax.numpy as jnp
from jax import lax
from jax.experimental import pallas as pl
from jax.experimental.pallas import tpu as pltpu
from jax.experimental.pallas import tpu_sc as plsc

E_ = 8
K_ = 2
H_ = 768
FFN_ = 3072
S_ = 2048
NPAIR = S_ * K_

M_ = 128
P_ = NPAIR + E_ * M_
NB_ = P_ // M_

NC_ = 2
NS_ = 16
NW_ = NC_ * NS_
CH_ = NPAIR // NW_
TPW_ = S_ // NW_
SUB_ = 32


def _router_body(x_ref, wr_ref, br_ref, bb_ref,
                 dst_ref, p_ref, bexp_ref, nv_ref):
    x = x_ref[...]
    wr = wr_ref[...]
    scores = lax.dot_general(x, wr, (((1,), (1,)), ((), ())),
                             preferred_element_type=jnp.float32)
    scores = scores + br_ref[...]
    m = jnp.max(scores, axis=1, keepdims=True)
    ex = jnp.exp(scores - m)
    probs = ex / jnp.sum(ex, axis=1, keepdims=True)

    eidx = lax.broadcasted_iota(jnp.int32, (S_, E_), 1)
    neg = jnp.float32(-1e30)

    sb = probs + bb_ref[...]
    m1 = jnp.max(sb, axis=1, keepdims=True)
    id1 = jnp.min(jnp.where(sb == m1, eidx, E_), axis=1, keepdims=True)
    sb2 = jnp.where(eidx == id1, neg, sb)
    m2 = jnp.max(sb2, axis=1, keepdims=True)
    id2 = jnp.min(jnp.where(sb2 == m2, eidx, E_), axis=1, keepdims=True)

    q1 = jnp.max(probs, axis=1, keepdims=True)
    iq1 = jnp.min(jnp.where(probs == q1, eidx, E_), axis=1, keepdims=True)
    pr2 = jnp.where(eidx == iq1, neg, probs)
    q2 = jnp.max(pr2, axis=1, keepdims=True)
    psum = q1 + q2
    p_ref[...] = jnp.concatenate([q1 / psum, q2 / psum], axis=1)

    c1 = (eidx == id1).astype(jnp.int32)
    c2 = (eidx == id2).astype(jnp.int32)
    ctot = c1 + c2
    acc = ctot
    d = 1
    while d < S_:
        shifted = jnp.concatenate(
            [jnp.zeros((d, E_), jnp.int32), acc[:S_ - d]], axis=0)
        acc = acc + shifted
        d *= 2
    excl = acc - ctot
    cnt = acc[S_ - 1:S_]

    pc = ((cnt + (M_ - 1)) >> 7) << 7
    bincl = pc
    d = 1
    while d < E_:
        bincl = bincl + jnp.concatenate(
            [jnp.zeros((1, d), jnp.int32), bincl[:, :E_ - d]], axis=1)
        d *= 2
    base = bincl - pc
    total = jnp.max(bincl, axis=1, keepdims=True)

    dst1 = jnp.sum((base + excl) * c1, axis=1, keepdims=True)
    dst2 = jnp.sum((base + excl) * c2, axis=1, keepdims=True)
    dst_ref[...] = jnp.concatenate([dst1, dst2], axis=1)

    rowstart = lax.broadcasted_iota(jnp.int32, (NB_, 1), 0) * M_
    bexp_ref[...] = jnp.sum((base <= rowstart).astype(jnp.int32),
                            axis=1, keepdims=True) - 1
    nv_ref[...] = total >> 7


def _router(x2, Wr, br, b_buf):
    return pl.pallas_call(
        _router_body,
        out_shape=[
            jax.ShapeDtypeStruct((S_, K_), jnp.int32),
            jax.ShapeDtypeStruct((S_, K_), jnp.float32),
            jax.ShapeDtypeStruct((NB_, 1), jnp.int32),
            jax.ShapeDtypeStruct((1, 1), jnp.int32),
        ],
    )(x2, Wr, br.reshape(1, E_), b_buf.reshape(1, E_))


_sc_cache = {}


def _dispatch_kernel():
    if "dispatch" in _sc_cache:
        return _sc_cache["dispatch"]

    @functools.partial(
        pl.kernel,
        out_type=jax.ShapeDtypeStruct((P_, H_), jnp.float32),
        mesh=plsc.VectorSubcoreMesh(core_axis_name="c", subcore_axis_name="s"),
        scratch_types=[
            pltpu.VMEM((CH_,), jnp.int32),
            pltpu.VMEM((CH_,), jnp.int32),
            pltpu.VMEM((CH_, H_), jnp.float32),
            pltpu.SemaphoreType.DMA,
            pltpu.SemaphoreType.DMA,
        ],
    )
    def _dispatch(x_hbm, tok_hbm, dst_hbm, xs_hbm, tok_v, dst_v, rows_v,
                  sem_g, sem_s):
        wid = lax.axis_index("s") * NC_ + lax.axis_index("c")
        b0 = wid * CH_
        pltpu.sync_copy(tok_hbm.at[pl.ds(b0, CH_)], tok_v)
        pltpu.sync_copy(dst_hbm.at[pl.ds(b0, CH_)], dst_v)
        pltpu.async_copy(x_hbm.at[tok_v], rows_v, sem_g).wait()
        pltpu.async_copy(rows_v, xs_hbm.at[dst_v], sem_s).wait()

    _sc_cache["dispatch"] = _dispatch
    return _dispatch


def _ffn_body(bexp_ref, nv_ref, xs_ref, w1_ref, b1_ref, w2_ref, b2_ref,
              out_ref):
    b = pl.program_id(0)

    @pl.when(b < nv_ref[0])
    def _():
        xb = xs_ref[...].astype(jnp.bfloat16)
        w1 = w1_ref[0]
        h = lax.dot_general(xb, w1, (((1,), (1,)), ((), ())),
                            preferred_element_type=jnp.float32)
        h = h + b1_ref[0]
        h = 0.5 * h * (1.0 + lax.erf(h * jnp.float32(0.7071067811865476)))
        w2 = w2_ref[0]
        y = lax.dot_general(h.astype(jnp.bfloat16), w2,
                            (((1,), (1,)), ((), ())),
                            preferred_element_type=jnp.float32)
        out_ref[...] = y + b2_ref[0]


def _ffn(bexp, nv, xs, W1b, b1, W2b, b2):
    grid_spec = pltpu.PrefetchScalarGridSpec(
        num_scalar_prefetch=2,
        grid=(NB_,),
        in_specs=[
            pl.BlockSpec((M_, H_), lambda b, be, nv: (b, 0)),
            pl.BlockSpec((1, FFN_, H_), lambda b, be, nv: (be[b], 0, 0)),
            pl.BlockSpec((1, 1, FFN_), lambda b, be, nv: (be[b], 0, 0)),
            pl.BlockSpec((1, H_, FFN_), lambda b, be, nv: (be[b], 0, 0)),
            pl.BlockSpec((1, 1, H_), lambda b, be, nv: (be[b], 0, 0)),
        ],
        out_specs=pl.BlockSpec(
            (M_, H_), lambda b, be, nv: (jnp.where(b < nv[0], b, NB_), 0)),
    )
    return pl.pallas_call(
        _ffn_body,
        grid_spec=grid_spec,
        out_shape=jax.ShapeDtypeStruct(((NB_ + 1) * M_, H_), jnp.float32),
    )(bexp, nv, xs, W1b, b1, W2b, b2)


def _combine_kernel():
    if "combine" in _sc_cache:
        return _sc_cache["combine"]

    @functools.partial(
        pl.kernel,
        out_type=jax.ShapeDtypeStruct((S_, H_), jnp.float32),
        mesh=plsc.VectorSubcoreMesh(core_axis_name="c", subcore_axis_name="s"),
        scratch_types=[
            pltpu.VMEM((2 * SUB_,), jnp.int32),
            pltpu.VMEM((2 * SUB_, 16), jnp.float32),
            pltpu.VMEM((2 * SUB_, H_), jnp.float32),
            pltpu.VMEM((SUB_, H_), jnp.float32),
            pltpu.SemaphoreType.DMA,
        ],
    )
    def _combine(y_hbm, dst_hbm, prep_hbm, out_hbm, dst_v, p_v, rows_v, out_v,
                 sem):
        wid = lax.axis_index("s") * NC_ + lax.axis_index("c")
        for it in range(TPW_ // SUB_):
            t0 = wid * TPW_ + it * SUB_
            pltpu.sync_copy(dst_hbm.at[pl.ds(2 * t0, 2 * SUB_)], dst_v)
            pltpu.sync_copy(prep_hbm.at[pl.ds(2 * t0, 2 * SUB_)], p_v)
            pltpu.async_copy(y_hbm.at[dst_v], rows_v, sem).wait()

            def tloop(t, carry):
                p0 = p_v[2 * t]
                p1 = p_v[2 * t + 1]
                for c in range(H_ // 16):
                    sl = pl.ds(c * 16, 16)
                    out_v[t, sl] = p0 * rows_v[2 * t, sl] + \
                        p1 * rows_v[2 * t + 1, sl]
                return carry

            lax.fori_loop(0, SUB_, tloop, 0)
            pltpu.sync_copy(out_v, out_hbm.at[pl.ds(t0, SUB_)])

    _sc_cache["combine"] = _combine
    return _combine


def kernel(x, Wr, br, W1, b1, W2, b2, b_buf):
    b, s, h = x.shape
    x2 = x.reshape(S_, H_)
    dst, p12, bexp, nv = _router(x2, Wr, br, b_buf)

    dst_flat = dst.reshape(NPAIR)
    tok = jnp.repeat(jnp.arange(S_, dtype=jnp.int32), K_)
    p_rep = jnp.broadcast_to(p12.reshape(NPAIR, 1), (NPAIR, 16))

    xs = _dispatch_kernel()(x2, tok, dst_flat)
    y = _ffn(bexp.reshape(NB_), nv.reshape(1), xs,
             W1.astype(jnp.bfloat16), b1.reshape(E_, 1, FFN_),
             W2.astype(jnp.bfloat16), b2.reshape(E_, 1, H_))
    out = _combine_kernel()(y, dst_flat, p_rep)
    return out.reshape(b, s, h)


# speedup vs baseline: 2.8039x; 1.1866x over previous
import functools

import jax
import jax.numpy as jnp
from jax import lax
from jax.experimental import pallas as pl
from jax.experimental.pallas import tpu as pltpu
from jax.experimental.pallas import tpu_sc as plsc

E_ = 8
K_ = 2
H_ = 768
FFN_ = 3072
S_ = 2048
NPAIR = S_ * K_

M_ = 128
P_ = NPAIR + E_ * M_
NB_ = P_ // M_

NC_ = 2
NS_ = 16
NW_ = NC_ * NS_
CH_ = NPAIR // NW_
TPW_ = S_ // NW_
SUB_ = 32


def _router_body(x_ref, wr_ref, br_ref, bb_ref,
                 dst_ref, p_ref, bexp_ref, nv_ref):
    x = x_ref[...]
    wr = wr_ref[...]
    scores = lax.dot_general(x, wr, (((1,), (1,)), ((), ())),
                             preferred_element_type=jnp.float32)
    scores = scores + br_ref[...]
    m = jnp.max(scores, axis=1, keepdims=True)
    ex = jnp.exp(scores - m)
    probs = ex / jnp.sum(ex, axis=1, keepdims=True)

    eidx = lax.broadcasted_iota(jnp.int32, (S_, E_), 1)
    neg = jnp.float32(-1e30)

    sb = probs + bb_ref[...]
    m1 = jnp.max(sb, axis=1, keepdims=True)
    id1 = jnp.min(jnp.where(sb == m1, eidx, E_), axis=1, keepdims=True)
    sb2 = jnp.where(eidx == id1, neg, sb)
    m2 = jnp.max(sb2, axis=1, keepdims=True)
    id2 = jnp.min(jnp.where(sb2 == m2, eidx, E_), axis=1, keepdims=True)

    q1 = jnp.max(probs, axis=1, keepdims=True)
    iq1 = jnp.min(jnp.where(probs == q1, eidx, E_), axis=1, keepdims=True)
    pr2 = jnp.where(eidx == iq1, neg, probs)
    q2 = jnp.max(pr2, axis=1, keepdims=True)
    psum = q1 + q2
    p_ref[...] = jnp.concatenate([q1 / psum, q2 / psum], axis=1)

    c1 = (eidx == id1).astype(jnp.int32)
    c2 = (eidx == id2).astype(jnp.int32)
    ctot = c1 + c2
    acc = ctot
    d = 1
    while d < S_:
        shifted = jnp.concatenate(
            [jnp.zeros((d, E_), jnp.int32), acc[:S_ - d]], axis=0)
        acc = acc + shifted
        d *= 2
    excl = acc - ctot
    cnt = acc[S_ - 1:S_]

    pc = ((cnt + (M_ - 1)) >> 7) << 7
    bincl = pc
    d = 1
    while d < E_:
        bincl = bincl + jnp.concatenate(
            [jnp.zeros((1, d), jnp.int32), bincl[:, :E_ - d]], axis=1)
        d *= 2
    base = bincl - pc
    total = jnp.max(bincl, axis=1, keepdims=True)

    dst1 = jnp.sum((base + excl) * c1, axis=1, keepdims=True)
    dst2 = jnp.sum((base + excl) * c2, axis=1, keepdims=True)
    dst_ref[...] = jnp.concatenate([dst1, dst2], axis=1)

    rowstart = lax.broadcasted_iota(jnp.int32, (NB_, 1), 0) * M_
    bexp_ref[...] = jnp.sum((base <= rowstart).astype(jnp.int32),
                            axis=1, keepdims=True) - 1
    nv_ref[...] = total >> 7


def _router(x2, Wr, br, b_buf):
    return pl.pallas_call(
        _router_body,
        out_shape=[
            jax.ShapeDtypeStruct((S_, K_), jnp.int32),
            jax.ShapeDtypeStruct((S_, K_), jnp.float32),
            jax.ShapeDtypeStruct((NB_, 1), jnp.int32),
            jax.ShapeDtypeStruct((1, 1), jnp.int32),
        ],
    )(x2, Wr, br.reshape(1, E_), b_buf.reshape(1, E_))


_sc_cache = {}


def _dispatch_kernel():
    if "dispatch" in _sc_cache:
        return _sc_cache["dispatch"]

    TW = S_ // NW_

    @functools.partial(
        pl.kernel,
        out_type=jax.ShapeDtypeStruct((P_, H_), jnp.float32),
        mesh=plsc.VectorSubcoreMesh(core_axis_name="c", subcore_axis_name="s"),
        scratch_types=[
            pltpu.VMEM((TW,), jnp.int32),
            pltpu.VMEM((TW,), jnp.int32),
            pltpu.VMEM((TW, H_), jnp.float32),
            pltpu.SemaphoreType.DMA,
            pltpu.SemaphoreType.DMA,
        ],
    )
    def _dispatch(x_hbm, d0_hbm, d1_hbm, xs_hbm, d0_v, d1_v, rows_v,
                  sem_g, sem_s):
        wid = lax.axis_index("s") * NC_ + lax.axis_index("c")
        t0 = wid * TW
        pltpu.sync_copy(d0_hbm.at[pl.ds(t0, TW)], d0_v)
        pltpu.sync_copy(d1_hbm.at[pl.ds(t0, TW)], d1_v)
        pltpu.async_copy(x_hbm.at[pl.ds(t0, TW)], rows_v, sem_g).wait()
        cp = pltpu.async_copy(rows_v, xs_hbm.at[d0_v], sem_s)
        pltpu.async_copy(rows_v, xs_hbm.at[d1_v], sem_s).wait()
        cp.wait()

    _sc_cache["dispatch"] = _dispatch
    return _dispatch


def _ffn_body(bexp_ref, nv_ref, xs_ref, w1_ref, b1_ref, w2_ref, b2_ref,
              out_ref):
    b = pl.program_id(0)

    @pl.when(b < nv_ref[0])
    def _():
        xb = xs_ref[...].astype(jnp.bfloat16)
        w1 = w1_ref[0].astype(jnp.bfloat16)
        h = lax.dot_general(xb, w1, (((1,), (1,)), ((), ())),
                            preferred_element_type=jnp.float32)
        h = h + b1_ref[0]
        h = 0.5 * h * (1.0 + lax.erf(h * jnp.float32(0.7071067811865476)))
        w2 = w2_ref[0].astype(jnp.bfloat16)
        y = lax.dot_general(h.astype(jnp.bfloat16), w2,
                            (((1,), (1,)), ((), ())),
                            preferred_element_type=jnp.float32)
        out_ref[...] = y + b2_ref[0]


def _ffn(bexp, nv, xs, W1b, b1, W2b, b2):
    grid_spec = pltpu.PrefetchScalarGridSpec(
        num_scalar_prefetch=2,
        grid=(NB_,),
        in_specs=[
            pl.BlockSpec((M_, H_), lambda b, be, nv: (b, 0)),
            pl.BlockSpec((1, FFN_, H_), lambda b, be, nv: (be[b], 0, 0)),
            pl.BlockSpec((1, 1, FFN_), lambda b, be, nv: (be[b], 0, 0)),
            pl.BlockSpec((1, H_, FFN_), lambda b, be, nv: (be[b], 0, 0)),
            pl.BlockSpec((1, 1, H_), lambda b, be, nv: (be[b], 0, 0)),
        ],
        out_specs=pl.BlockSpec(
            (M_, H_), lambda b, be, nv: (jnp.where(b < nv[0], b, NB_), 0)),
    )
    return pl.pallas_call(
        _ffn_body,
        grid_spec=grid_spec,
        out_shape=jax.ShapeDtypeStruct(((NB_ + 1) * M_, H_), jnp.float32),
    )(bexp, nv, xs, W1b, b1, W2b, b2)


def _combine_kernel():
    if "combine" in _sc_cache:
        return _sc_cache["combine"]

    @functools.partial(
        pl.kernel,
        out_type=jax.ShapeDtypeStruct((S_, H_), jnp.float32),
        mesh=plsc.VectorSubcoreMesh(core_axis_name="c", subcore_axis_name="s"),
        scratch_types=[
            pltpu.VMEM((2 * SUB_,), jnp.int32),
            pltpu.VMEM((2 * SUB_, 16), jnp.float32),
            pltpu.VMEM((2 * SUB_, H_), jnp.float32),
            pltpu.VMEM((SUB_, H_), jnp.float32),
            pltpu.SemaphoreType.DMA,
        ],
    )
    def _combine(y_hbm, dst_hbm, prep_hbm, out_hbm, dst_v, p_v, rows_v, out_v,
                 sem):
        wid = lax.axis_index("s") * NC_ + lax.axis_index("c")
        for it in range(TPW_ // SUB_):
            t0 = wid * TPW_ + it * SUB_
            pltpu.sync_copy(dst_hbm.at[pl.ds(2 * t0, 2 * SUB_)], dst_v)
            pltpu.sync_copy(prep_hbm.at[pl.ds(2 * t0, 2 * SUB_)], p_v)
            pltpu.async_copy(y_hbm.at[dst_v], rows_v, sem).wait()

            def tloop(t, carry):
                p0 = p_v[2 * t]
                p1 = p_v[2 * t + 1]
                for c in range(H_ // 16):
                    sl = pl.ds(c * 16, 16)
                    out_v[t, sl] = p0 * rows_v[2 * t, sl] + \
                        p1 * rows_v[2 * t + 1, sl]
                return carry

            lax.fori_loop(0, SUB_, tloop, 0)
            pltpu.sync_copy(out_v, out_hbm.at[pl.ds(t0, SUB_)])

    _sc_cache["combine"] = _combine
    return _combine


def kernel(x, Wr, br, W1, b1, W2, b2, b_buf):
    b, s, h = x.shape
    x2 = x.reshape(S_, H_)
    dst, p12, bexp, nv = _router(x2, Wr, br, b_buf)

    dst_flat = dst.reshape(NPAIR)
    p_rep = jnp.broadcast_to(p12.reshape(NPAIR, 1), (NPAIR, 16))

    xs = _dispatch_kernel()(x2, dst[:, 0], dst[:, 1])
    y = _ffn(bexp.reshape(NB_), nv.reshape(1), xs,
             W1, b1.reshape(E_, 1, FFN_), W2, b2.reshape(E_, 1, H_))
    out = _combine_kernel()(y, dst_flat, p_rep)
    return out.reshape(b, s, h)
